# baseline (speedup 1.0000x reference)
import jax
import jax.numpy as jnp
from jax.experimental import pallas as pl
from jax.experimental.pallas import tpu as pltpu

K_NN = 20


N_SLOTS = 5
LANES = 128


def _knn_feat_kernel(x_cur_ref, x_all_cur_ref, x_prev_ref, xhi_prev_ref,
                     xlo_prev_ref, w1_ref, wd_ref, bias_ref, out_ref,
                     dbuf_ref, mref):
    i = pl.program_id(0)
    nsteps = pl.num_programs(0) - 1
    neg_inf = jnp.float32(-jnp.inf)

    @pl.when(i < nsteps)
    def _compute_dist():
        x_blk = x_cur_ref[...]
        x_all = x_all_cur_ref[...]
        c2 = jnp.sum(x_all * x_all, axis=0, keepdims=True)
        d = 2.0 * jax.lax.dot_general(
            x_blk, x_all, (((0,), (0,)), ((), ())),
            preferred_element_type=jnp.float32)
        dbuf_ref[i % 2] = d - c2

    @pl.when(i > 0)
    def _select_and_project():
        d = dbuf_ref[(i - 1) % 2]
        N = d.shape[1]
        s = [None] * N_SLOTS
        for t in range(N // LANES):
            v = d[:, t * LANES:(t + 1) * LANES]
            for q in range(N_SLOTS):
                if s[q] is None:
                    s[q] = v
                    break
                hi = jnp.maximum(s[q], v)
                v = jnp.minimum(s[q], v)
                s[q] = hi
        for q in range(N_SLOTS):
            if s[q] is None:
                s[q] = jnp.full_like(s[0], neg_inf)

        m = jnp.max(s[0], axis=1, keepdims=True)
        for j in range(K_NN - 1):
            c = s[0] == m
            for q in range(N_SLOTS - 1):
                s[q] = jnp.where(c, s[q + 1], s[q])
            s[N_SLOTS - 1] = jnp.where(c, neg_inf, s[N_SLOTS - 1])
            m = jnp.max(s[0], axis=1, keepdims=True)

        cnt = jnp.sum((d >= m).astype(jnp.float32), axis=1)
        bad = jnp.max(jnp.abs(cnt - jnp.float32(K_NN))) > 0.0
        mref[...] = m

        @pl.when(bad)
        def _full_chain():
            mm = jnp.max(d, axis=1, keepdims=True)
            for _ in range(K_NN - 1):
                mm = jnp.max(jnp.where(d < mm, d, neg_inf),
                             axis=1, keepdims=True)
            mref[...] = mm

        mask = (d >= mref[...]).astype(jnp.bfloat16)

        dims = (((1,), (1,)), ((), ()))
        g = jax.lax.dot_general(xhi_prev_ref[...], mask, dims,
                                preferred_element_type=jnp.float32)
        g = g + jax.lax.dot_general(xlo_prev_ref[...], mask, dims,
                                    preferred_element_type=jnp.float32)
        g = g * jnp.float32(1.0 / K_NN)

        o = jax.lax.dot_general(w1_ref[...], g, (((1,), (0,)), ((), ())),
                                preferred_element_type=jnp.float32)
        o = o + jax.lax.dot_general(wd_ref[...], x_prev_ref[...],
                                    (((1,), (0,)), ((), ())),
                                    preferred_element_type=jnp.float32)
        out_ref[...] = o + bias_ref[...]


def kernel(x, W, b):
    B, C, N = x.shape
    O = W.shape[0]
    Wm = W[:, :, 0, 0]
    w1 = Wm[:, :C]
    wd = Wm[:, C:] - w1
    bias = b[:, None]
    x_hi = x.astype(jnp.bfloat16)
    x_lo = (x - x_hi.astype(jnp.float32)).astype(jnp.bfloat16)

    R = min(512, N)
    nb = N // R
    nblocks = B * nb
    grid = (nblocks + 1,)

    def cur_blk(t):
        f = jnp.minimum(t, nblocks - 1)
        return f // nb, 0, f % nb

    def prev_blk(t):
        g = jnp.maximum(t - 1, 0)
        return g // nb, 0, g % nb

    return pl.pallas_call(
        _knn_feat_kernel,
        grid=grid,
        in_specs=[
            pl.BlockSpec((None, C, R), cur_blk),
            pl.BlockSpec((None, C, N), lambda t: (jnp.minimum(t, nblocks - 1) // nb, 0, 0)),
            pl.BlockSpec((None, C, R), prev_blk),
            pl.BlockSpec((None, C, N), lambda t: (jnp.maximum(t - 1, 0) // nb, 0, 0)),
            pl.BlockSpec((None, C, N), lambda t: (jnp.maximum(t - 1, 0) // nb, 0, 0)),
            pl.BlockSpec((O, C), lambda t: (0, 0)),
            pl.BlockSpec((O, C), lambda t: (0, 0)),
            pl.BlockSpec((O, 1), lambda t: (0, 0)),
        ],
        out_specs=pl.BlockSpec((None, O, R), prev_blk),
        out_shape=jax.ShapeDtypeStruct((B, O, N), jnp.float32),
        scratch_shapes=[pltpu.VMEM((2, R, N), jnp.float32),
                        pltpu.VMEM((R, 1), jnp.float32)],
    )(x, x, x, x_hi, x_lo, w1, wd, bias)


# speedup vs baseline: 1.1662x; 1.1662x over previous
import jax
import jax.numpy as jnp
from jax.experimental import pallas as pl
from jax.experimental.pallas import tpu as pltpu

K_NN = 20
N_SLOTS = 5
LANES = 128


def _knn_feat_kernel(x_cur_ref, x_all_cur_ref, x_pp_ref, x_all_pp_ref,
                     w1_ref, wd_ref, bias_ref, out_ref,
                     dbuf_ref, maskbuf_ref):
    i = pl.program_id(0)
    nblocks = pl.num_programs(0) - 2
    neg_inf = jnp.float32(-jnp.inf)

    @pl.when(i < nblocks)
    def _compute_dist():
        x_blk = x_cur_ref[...]
        x_all = x_all_cur_ref[...]
        c2 = jnp.sum(x_all * x_all, axis=0, keepdims=True)
        d = 2.0 * jax.lax.dot_general(
            x_blk, x_all, (((0,), (0,)), ((), ())),
            preferred_element_type=jnp.float32)
        dbuf_ref[i % 2] = d - c2

    @pl.when(jnp.logical_and(i > 0, i <= nblocks))
    def _select():
        d = dbuf_ref[(i - 1) % 2]
        N = d.shape[1]
        s = [None] * N_SLOTS
        for t in range(N // LANES):
            v = d[:, t * LANES:(t + 1) * LANES]
            for q in range(N_SLOTS):
                if s[q] is None:
                    s[q] = v
                    break
                hi = jnp.maximum(s[q], v)
                v = jnp.minimum(s[q], v)
                s[q] = hi
        for q in range(N_SLOTS):
            if s[q] is None:
                s[q] = jnp.full_like(s[0], neg_inf)

        m = jnp.max(s[0], axis=1, keepdims=True)
        for _ in range(K_NN - 1):
            c = s[0] == m
            for q in range(N_SLOTS - 1):
                s[q] = jnp.where(c, s[q + 1], s[q])
            s[N_SLOTS - 1] = jnp.where(c, neg_inf, s[N_SLOTS - 1])
            m = jnp.max(s[0], axis=1, keepdims=True)

        mask = (d >= m).astype(jnp.float32)
        maskbuf_ref[(i - 1) % 2] = mask
        cnt = jnp.sum(mask, axis=1)
        bad = jnp.max(jnp.abs(cnt - jnp.float32(K_NN))) > 0.0

        @pl.when(bad)
        def _full_chain():
            mm = jnp.max(d, axis=1, keepdims=True)
            for _ in range(K_NN - 1):
                mm = jnp.max(jnp.where(d < mm, d, neg_inf),
                             axis=1, keepdims=True)
            maskbuf_ref[(i - 1) % 2] = (d >= mm).astype(jnp.float32)

    @pl.when(i > 1)
    def _project():
        mask = maskbuf_ref[(i - 2) % 2]
        g = jax.lax.dot_general(x_all_pp_ref[...], mask,
                                (((1,), (1,)), ((), ())),
                                preferred_element_type=jnp.float32)
        g = g * jnp.float32(1.0 / K_NN)

        o = jax.lax.dot_general(w1_ref[...], g, (((1,), (0,)), ((), ())),
                                preferred_element_type=jnp.float32)
        o = o + jax.lax.dot_general(wd_ref[...], x_pp_ref[...],
                                    (((1,), (0,)), ((), ())),
                                    preferred_element_type=jnp.float32)
        out_ref[...] = o + bias_ref[...]


def kernel(x, W, b):
    B, C, N = x.shape
    O = W.shape[0]
    Wm = W[:, :, 0, 0]
    w1 = Wm[:, :C]
    wd = Wm[:, C:] - w1
    bias = b[:, None]

    R = min(512, N)
    nb = N // R
    nblocks = B * nb
    grid = (nblocks + 2,)

    def cur_blk(t):
        f = jnp.minimum(t, nblocks - 1)
        return f // nb, 0, f % nb

    def pp_blk(t):
        g = jnp.maximum(t - 2, 0)
        return g // nb, 0, g % nb

    return pl.pallas_call(
        _knn_feat_kernel,
        grid=grid,
        in_specs=[
            pl.BlockSpec((None, C, R), cur_blk),
            pl.BlockSpec((None, C, N), lambda t: (jnp.minimum(t, nblocks - 1) // nb, 0, 0)),
            pl.BlockSpec((None, C, R), pp_blk),
            pl.BlockSpec((None, C, N), lambda t: (jnp.maximum(t - 2, 0) // nb, 0, 0)),
            pl.BlockSpec((O, C), lambda t: (0, 0)),
            pl.BlockSpec((O, C), lambda t: (0, 0)),
            pl.BlockSpec((O, 1), lambda t: (0, 0)),
        ],
        out_specs=pl.BlockSpec((None, O, R), pp_blk),
        out_shape=jax.ShapeDtypeStruct((B, O, N), jnp.float32),
        scratch_shapes=[pltpu.VMEM((2, R, N), jnp.float32),
                        pltpu.VMEM((2, R, N), jnp.float32)],
    )(x, x, x, x, w1, wd, bias)


# speedup vs baseline: 1.2202x; 1.0463x over previous
import jax
import jax.numpy as jnp
from jax.experimental import pallas as pl
from jax.experimental.pallas import tpu as pltpu

K_NN = 20
N_SLOTS = 4
LANES = 128


def _knn_feat_kernel(x_cur_ref, x_all_cur_ref, x_pp_ref, x_all_pp_ref,
                     w1_ref, wd_ref, bias_ref, out_ref,
                     dbuf_ref, maskbuf_ref):
    i = pl.program_id(0)
    nblocks = pl.num_programs(0) - 2
    neg_inf = jnp.float32(-jnp.inf)

    @pl.when(i < nblocks)
    def _compute_dist():
        x_blk = x_cur_ref[...]
        x_all = x_all_cur_ref[...]
        c2 = jnp.sum(x_all * x_all, axis=0, keepdims=True)
        d = 2.0 * jax.lax.dot_general(
            x_blk, x_all, (((0,), (0,)), ((), ())),
            preferred_element_type=jnp.float32)
        dbuf_ref[i % 2] = d - c2

    @pl.when(jnp.logical_and(i > 0, i <= nblocks))
    def _select():
        d = dbuf_ref[(i - 1) % 2]
        N = d.shape[1]
        s = [None] * N_SLOTS
        for t in range(N // LANES):
            v = d[:, t * LANES:(t + 1) * LANES]
            for q in range(N_SLOTS):
                if s[q] is None:
                    s[q] = v
                    break
                hi = jnp.maximum(s[q], v)
                v = jnp.minimum(s[q], v)
                s[q] = hi
        for q in range(N_SLOTS):
            if s[q] is None:
                s[q] = jnp.full_like(s[0], neg_inf)

        m = jnp.max(s[0], axis=1, keepdims=True)
        for _ in range(K_NN - 1):
            c = s[0] == m
            for q in range(N_SLOTS - 1):
                s[q] = jnp.where(c, s[q + 1], s[q])
            s[N_SLOTS - 1] = jnp.where(c, neg_inf, s[N_SLOTS - 1])
            m = jnp.max(s[0], axis=1, keepdims=True)

        mask = (d >= m).astype(jnp.float32)
        maskbuf_ref[(i - 1) % 2] = mask
        cnt = jnp.sum(mask, axis=1)
        bad = jnp.max(jnp.abs(cnt - jnp.float32(K_NN))) > 0.0

        @pl.when(bad)
        def _full_chain():
            mm = jnp.max(d, axis=1, keepdims=True)
            for _ in range(K_NN - 1):
                mm = jnp.max(jnp.where(d < mm, d, neg_inf),
                             axis=1, keepdims=True)
            maskbuf_ref[(i - 1) % 2] = (d >= mm).astype(jnp.float32)

    @pl.when(i > 1)
    def _project():
        mask = maskbuf_ref[(i - 2) % 2]
        g = jax.lax.dot_general(x_all_pp_ref[...], mask,
                                (((1,), (1,)), ((), ())),
                                preferred_element_type=jnp.float32)
        g = g * jnp.float32(1.0 / K_NN)

        o = jax.lax.dot_general(w1_ref[...], g, (((1,), (0,)), ((), ())),
                                preferred_element_type=jnp.float32)
        o = o + jax.lax.dot_general(wd_ref[...], x_pp_ref[...],
                                    (((1,), (0,)), ((), ())),
                                    preferred_element_type=jnp.float32)
        out_ref[...] = o + bias_ref[...]


def kernel(x, W, b):
    B, C, N = x.shape
    O = W.shape[0]
    Wm = W[:, :, 0, 0]
    w1 = Wm[:, :C]
    wd = Wm[:, C:] - w1
    bias = b[:, None]

    R = min(512, N)
    nb = N // R
    nblocks = B * nb
    grid = (nblocks + 2,)

    def cur_blk(t):
        f = jnp.minimum(t, nblocks - 1)
        return f // nb, 0, f % nb

    def pp_blk(t):
        g = jnp.maximum(t - 2, 0)
        return g // nb, 0, g % nb

    return pl.pallas_call(
        _knn_feat_kernel,
        grid=grid,
        in_specs=[
            pl.BlockSpec((None, C, R), cur_blk),
            pl.BlockSpec((None, C, N), lambda t: (jnp.minimum(t, nblocks - 1) // nb, 0, 0)),
            pl.BlockSpec((None, C, R), pp_blk),
            pl.BlockSpec((None, C, N), lambda t: (jnp.maximum(t - 2, 0) // nb, 0, 0)),
            pl.BlockSpec((O, C), lambda t: (0, 0)),
            pl.BlockSpec((O, C), lambda t: (0, 0)),
            pl.BlockSpec((O, 1), lambda t: (0, 0)),
        ],
        out_specs=pl.BlockSpec((None, O, R), pp_blk),
        out_shape=jax.ShapeDtypeStruct((B, O, N), jnp.float32),
        scratch_shapes=[pltpu.VMEM((2, R, N), jnp.float32),
                        pltpu.VMEM((2, R, N), jnp.float32)],
    )(x, x, x, x, w1, wd, bias)
